# Initial kernel scaffold; baseline (speedup 1.0000x reference)
import jax
import jax.numpy as jnp
from jax.experimental import pallas as pl


def kernel(scalars, x, params):
    raise NotImplementedError("write your pallas kernel here")


# speedup vs baseline: 8.4432x; 8.4432x over previous
import jax
import jax.numpy as jnp
from jax.experimental import pallas as pl
from jax.experimental.pallas import tpu as pltpu

_NS = 7
_H = 72
_NC = 2
_NL = 6
_CW = 0.001
_B = 32
_N = 64
_IBS = 16
_NBLK = _N // _IBS
_EPS = 1e-5
_ECNT = float(_B * _N * (_N - 1))


def _psi(t):
    return jnp.sign(t) * jnp.log(jnp.abs(t) + 1.0)


def _relu(t):
    return jnp.maximum(t, 0.0)


def _dot(a, b):
    return jnp.dot(a, b, preferred_element_type=jnp.float32)


def _lorentz_body(scalars_ref, x_ref, emb_W_ref, emb_b_ref,
                  eW1_ref, eg1_ref, ebe1_ref, eW2_ref, eb2_ref,
                  mW_ref, mb_ref,
                  xW1_ref, xb1_ref, xW2_ref,
                  hW1_ref, hb1_ref, hg_ref, hbe_ref, hW2_ref, hb2_ref,
                  dW1_ref, db1_ref, dW2_ref, db2_ref,
                  out_ref,
                  h_ref, xc_ref, xci_ref, norms_ref, dots_ref, Ai_ref, Aj_ref,
                  s1_ref, s2_ref, sc_ref, sh_ref, hagg_ref, aggx_ref):
    l = pl.program_id(0)
    p = pl.program_id(1)
    k = pl.program_id(2)
    i0 = k * _IBS

    @pl.when(jnp.logical_and(l == 0, jnp.logical_and(p == 0, k == 0)))
    def _init():
        scal2 = scalars_ref[...].reshape(_B * _N, _NS)
        h_ref[...] = _dot(scal2, emb_W_ref[...]) + emb_b_ref[...]
        for c in range(4):
            xf = x_ref[..., c]
            xc_ref[c] = xf
            for kk in range(_NBLK):
                xci_ref[kk, c] = xf[:, kk * _IBS:(kk + 1) * _IBS]

    @pl.when(jnp.logical_and(p == 0, k == 0))
    def _layer_prologue():
        x0, x1, x2, x3 = xc_ref[0], xc_ref[1], xc_ref[2], xc_ref[3]
        G = (x0[:, :, None] * x0[:, None, :]
             - x1[:, :, None] * x1[:, None, :]
             - x2[:, :, None] * x2[:, None, :]
             - x3[:, :, None] * x3[:, None, :])
        dG = x0 * x0 - x1 * x1 - x2 * x2 - x3 * x3
        dots_ref[...] = _psi(G)
        norms_ref[...] = _psi(dG[:, :, None] + dG[:, None, :] - 2.0 * G)
        w1 = eW1_ref[l]
        h = h_ref[...]
        Ai_ref[...] = _dot(h, w1[0:_H, :]).reshape(_B, _N, _H)
        Aj_ref[...] = _dot(h, w1[_H:2 * _H, :]).reshape(_B, _N, _H)
        s1_ref[...] = jnp.zeros(s1_ref.shape, jnp.float32)
        s2_ref[...] = jnp.zeros(s2_ref.shape, jnp.float32)

    def m1_block():
        w1 = eW1_ref[l]
        w_n = w1[2 * _H, :].reshape(1, 1, 1, _H)
        w_d = w1[2 * _H + 1, :].reshape(1, 1, 1, _H)
        nb = norms_ref[:, pl.ds(i0, _IBS), :][..., None]
        db = dots_ref[:, pl.ds(i0, _IBS), :][..., None]
        ai = Ai_ref[:, pl.ds(i0, _IBS), :]
        return (ai[:, :, None, :] + Aj_ref[...][:, None, :, :]
                + nb * w_n + db * w_d)

    ib = jax.lax.broadcasted_iota(jnp.int32, (_IBS, _N), 0) + i0
    jb = jax.lax.broadcasted_iota(jnp.int32, (_IBS, _N), 1)
    msk = (ib != jb).astype(jnp.float32)[None, :, :, None]

    @pl.when(p == 0)
    def _stats_pass():
        m2 = (m1_block() * msk).reshape(_B * _IBS * _N, _H)
        s1_ref[...] += jnp.sum(m2, axis=0).reshape(1, _H)
        s2_ref[...] += jnp.sum(m2 * m2, axis=0).reshape(1, _H)

    @pl.when(jnp.logical_and(p == 1, k == 0))
    def _stats_finalize():
        mean = s1_ref[...] / _ECNT
        var = s2_ref[...] / _ECNT - mean * mean
        scv = eg1_ref[l].reshape(1, _H) / jnp.sqrt(var + _EPS)
        sc_ref[...] = scv
        sh_ref[...] = ebe1_ref[l].reshape(1, _H) - mean * scv

    @pl.when(p == 1)
    def _compute_pass():
        scv = sc_ref[...].reshape(1, 1, 1, _H)
        shv = sh_ref[...].reshape(1, 1, 1, _H)
        e = _relu(m1_block() * scv + shv).reshape(_B * _IBS * _N, _H)
        f = _relu(_dot(e, eW2_ref[l]) + eb2_ref[l].reshape(1, _H))
        wgt = jax.nn.sigmoid(_dot(f, mW_ref[l]) + mb_ref[l].reshape(1, 1))
        m = f * wgt
        m4 = m.reshape(_B, _IBS, _N, _H) * msk
        hagg_ref[:, pl.ds(i0, _IBS), :] = jnp.sum(m4, axis=2)

        @pl.when(l < _NL - 1)
        def _coord_update():
            t = _relu(_dot(m, xW1_ref[l]) + xb1_ref[l].reshape(1, _H))
            tsq = _dot(t, xW2_ref[l]).reshape(_B, _IBS, _N)
            for c in range(4):
                xd = (xci_ref[k, c][:, :, None]
                      - xc_ref[c][:, None, :])
                trans = jnp.clip(xd * tsq, -100.0, 100.0)
                aggx_ref[k, c] = jnp.sum(trans, axis=2) / float(_N - 1)

    @pl.when(jnp.logical_and(p == 1, k == _NBLK - 1))
    def _layer_epilogue():
        h = h_ref[...]
        hagg = hagg_ref[...].reshape(_B * _N, _H)
        scal2 = scalars_ref[...].reshape(_B * _N, _NS)
        hW1 = hW1_ref[l]
        hh1 = (_dot(h, hW1[0:_H, :]) + _dot(hagg, hW1[_H:2 * _H, :])
               + _dot(scal2, hW1[2 * _H:2 * _H + _NS, :])
               + hb1_ref[l].reshape(1, _H))
        mu = jnp.sum(hh1, axis=0).reshape(1, _H) / float(_B * _N)
        vr = (jnp.sum(hh1 * hh1, axis=0).reshape(1, _H) / float(_B * _N)
              - mu * mu)
        scn = hg_ref[l].reshape(1, _H) / jnp.sqrt(vr + _EPS)
        shn = hbe_ref[l].reshape(1, _H) - mu * scn
        hh = _relu(hh1 * scn + shn)
        hnew = h + _dot(hh, hW2_ref[l]) + hb2_ref[l].reshape(1, _H)
        h_ref[...] = hnew

        @pl.when(l < _NL - 1)
        def _apply_x():
            for c in range(4):
                agg = jnp.concatenate(
                    [aggx_ref[kk, c] for kk in range(_NBLK)], axis=-1)
                xf = xc_ref[c] + agg * _CW
                xc_ref[c] = xf
                for kk in range(_NBLK):
                    xci_ref[kk, c] = xf[:, kk * _IBS:(kk + 1) * _IBS]

        @pl.when(l == _NL - 1)
        def _decode():
            hm = jnp.sum(hnew.reshape(_B, _N, _H), axis=1) / float(_N)
            d1 = _relu(_dot(hm, dW1_ref[...]) + db1_ref[...])
            out_ref[...] = _dot(d1, dW2_ref[...]) + db2_ref[...]


def _const_spec(a):
    nd = a.ndim
    return pl.BlockSpec(a.shape, lambda l, p, k, _nd=nd: (0,) * _nd)


@jax.jit
def kernel(scalars, x, params):
    p = params
    args = (scalars, x,
            p['emb_W'], p['emb_b'].reshape(1, _H),
            p['phi_e_W1'], p['phi_e_g1'], p['phi_e_be1'],
            p['phi_e_W2'], p['phi_e_b2'],
            p['phi_m_W'], p['phi_m_b'],
            p['phi_x_W1'], p['phi_x_b1'], p['phi_x_W2'],
            p['phi_h_W1'], p['phi_h_b1'], p['phi_h_g'], p['phi_h_be'],
            p['phi_h_W2'], p['phi_h_b2'],
            p['dec_W1'], p['dec_b1'].reshape(1, _H),
            p['dec_W2'], p['dec_b2'].reshape(1, _NC))
    return pl.pallas_call(
        _lorentz_body,
        grid=(_NL, 2, _NBLK),
        in_specs=[_const_spec(a) for a in args],
        out_specs=pl.BlockSpec((_B, _NC), lambda l, p, k: (0, 0)),
        out_shape=jax.ShapeDtypeStruct((_B, _NC), jnp.float32),
        scratch_shapes=[
            pltpu.VMEM((_B * _N, _H), jnp.float32),
            pltpu.VMEM((4, _B, _N), jnp.float32),
            pltpu.VMEM((_NBLK, 4, _B, _IBS), jnp.float32),
            pltpu.VMEM((_B, _N, _N), jnp.float32),
            pltpu.VMEM((_B, _N, _N), jnp.float32),
            pltpu.VMEM((_B, _N, _H), jnp.float32),
            pltpu.VMEM((_B, _N, _H), jnp.float32),
            pltpu.VMEM((1, _H), jnp.float32),
            pltpu.VMEM((1, _H), jnp.float32),
            pltpu.VMEM((1, _H), jnp.float32),
            pltpu.VMEM((1, _H), jnp.float32),
            pltpu.VMEM((_B, _N, _H), jnp.float32),
            pltpu.VMEM((_NBLK, 4, _B, _IBS), jnp.float32),
        ],
        compiler_params=pltpu.CompilerParams(
            dimension_semantics=("arbitrary", "arbitrary", "arbitrary")),
    )(*args)


# speedup vs baseline: 11.6214x; 1.3764x over previous
import jax
import jax.numpy as jnp
from jax.experimental import pallas as pl
from jax.experimental.pallas import tpu as pltpu

_NS = 7
_H = 72
_NC = 2
_NL = 6
_CW = 0.001
_B = 32
_N = 64
_IBS = 16
_NBLK = _N // _IBS
_EPS = 1e-5
_ECNT = float(_B * _N * (_N - 1))


def _psi(t):
    return jnp.sign(t) * jnp.log(jnp.abs(t) + 1.0)


def _relu(t):
    return jnp.maximum(t, 0.0)


def _dot(a, b):
    return jnp.dot(a, b, preferred_element_type=jnp.float32)


def _lorentz_body(scalars_ref, x_ref, emb_W_ref, emb_b_ref,
                  eW1_ref, eg1_ref, ebe1_ref, eW2_ref, eb2_ref,
                  mW_ref, mb_ref,
                  xW1_ref, xb1_ref, xW2_ref,
                  hW1_ref, hb1_ref, hg_ref, hbe_ref, hW2_ref, hb2_ref,
                  dW1_ref, db1_ref, dW2_ref, db2_ref,
                  out_ref,
                  h_ref, xc_ref, xci_ref, norms_ref, dots_ref, Ai_ref, Aj_ref,
                  sc_ref, sh_ref, hagg_ref, aggx_ref):
    l = pl.program_id(0)
    k = pl.program_id(1)
    i0 = (k - 1) * _IBS

    @pl.when(jnp.logical_and(l == 0, k == 0))
    def _init():
        scal2 = scalars_ref[...].reshape(_B * _N, _NS)
        h_ref[...] = _dot(scal2, emb_W_ref[...]) + emb_b_ref[...]
        for c in range(4):
            xf = x_ref[..., c]
            xc_ref[c] = xf
            for kk in range(_NBLK):
                xci_ref[kk, c] = xf[:, kk * _IBS:(kk + 1) * _IBS]

    @pl.when(k == 0)
    def _layer_prologue():
        x0, x1, x2, x3 = xc_ref[0], xc_ref[1], xc_ref[2], xc_ref[3]
        G = (x0[:, :, None] * x0[:, None, :]
             - x1[:, :, None] * x1[:, None, :]
             - x2[:, :, None] * x2[:, None, :]
             - x3[:, :, None] * x3[:, None, :])
        dG = x0 * x0 - x1 * x1 - x2 * x2 - x3 * x3
        dots = _psi(G)
        norms = _psi(dG[:, :, None] + dG[:, None, :] - 2.0 * G)
        dots_ref[...] = dots
        norms_ref[...] = norms
        w1 = eW1_ref[l]
        h = h_ref[...]
        Ai = _dot(h, w1[0:_H, :]).reshape(_B, _N, _H)
        Aj = _dot(h, w1[_H:2 * _H, :]).reshape(_B, _N, _H)
        Ai_ref[...] = Ai
        Aj_ref[...] = Aj

        iN = jax.lax.broadcasted_iota(jnp.int32, (_N, _N), 0)
        jN = jax.lax.broadcasted_iota(jnp.int32, (_N, _N), 1)
        mskNN = (iN != jN).astype(jnp.float32)[None, :, :]
        nm = norms * mskNN
        dm = dots * mskNN
        w_n = w1[2 * _H, :].reshape(1, _H)
        w_d = w1[2 * _H + 1, :].reshape(1, _H)

        SA = jnp.sum(Ai, axis=1)
        SB = jnp.sum(Aj, axis=1)
        QA = jnp.sum(Ai * Ai, axis=1)
        QB = jnp.sum(Aj * Aj, axis=1)
        PAB = jnp.sum(Ai * Aj, axis=1)
        nrow = jnp.sum(nm, axis=2)
        ncol = jnp.sum(nm, axis=1)
        drow = jnp.sum(dm, axis=2)
        dcol = jnp.sum(dm, axis=1)
        Vn = jnp.sum(Ai * nrow[..., None], axis=1)
        VnB = jnp.sum(Aj * ncol[..., None], axis=1)
        Vd = jnp.sum(Ai * drow[..., None], axis=1)
        VdB = jnp.sum(Aj * dcol[..., None], axis=1)
        Sn = jnp.sum(nrow)
        Sd = jnp.sum(drow)
        Snn = jnp.sum(nm * nm)
        Sdd = jnp.sum(dm * dm)
        Snd = jnp.sum(nm * dm)

        s1 = (float(_N - 1) * jnp.sum(SA + SB, axis=0).reshape(1, _H)
              + Sn * w_n + Sd * w_d)
        s2 = (float(_N - 1) * jnp.sum(QA + QB, axis=0).reshape(1, _H)
              + Snn * (w_n * w_n) + Sdd * (w_d * w_d)
              + 2.0 * jnp.sum(SA * SB - PAB, axis=0).reshape(1, _H)
              + 2.0 * w_n * jnp.sum(Vn + VnB, axis=0).reshape(1, _H)
              + 2.0 * w_d * jnp.sum(Vd + VdB, axis=0).reshape(1, _H)
              + 2.0 * Snd * (w_n * w_d))
        mean = s1 / _ECNT
        var = s2 / _ECNT - mean * mean
        scv = eg1_ref[l].reshape(1, _H) / jnp.sqrt(var + _EPS)
        sc_ref[...] = scv
        sh_ref[...] = ebe1_ref[l].reshape(1, _H) - mean * scv

    @pl.when(k > 0)
    def _compute_pass():
        w1 = eW1_ref[l]
        w_n = w1[2 * _H, :].reshape(1, 1, 1, _H)
        w_d = w1[2 * _H + 1, :].reshape(1, 1, 1, _H)
        nb = norms_ref[:, pl.ds(i0, _IBS), :][..., None]
        db = dots_ref[:, pl.ds(i0, _IBS), :][..., None]
        ai = Ai_ref[:, pl.ds(i0, _IBS), :]
        m1 = (ai[:, :, None, :] + Aj_ref[...][:, None, :, :]
              + nb * w_n + db * w_d)

        ib = jax.lax.broadcasted_iota(jnp.int32, (_IBS, _N), 0) + i0
        jb = jax.lax.broadcasted_iota(jnp.int32, (_IBS, _N), 1)
        msk = (ib != jb).astype(jnp.float32)[None, :, :, None]

        scv = sc_ref[...].reshape(1, 1, 1, _H)
        shv = sh_ref[...].reshape(1, 1, 1, _H)
        e = _relu(m1 * scv + shv).reshape(_B * _IBS * _N, _H)
        f = _relu(_dot(e, eW2_ref[l]) + eb2_ref[l].reshape(1, _H))
        wgt = jax.nn.sigmoid(_dot(f, mW_ref[l]) + mb_ref[l].reshape(1, 1))
        m = f * wgt
        m4 = m.reshape(_B, _IBS, _N, _H) * msk
        hagg_ref[:, pl.ds(i0, _IBS), :] = jnp.sum(m4, axis=2)

        @pl.when(l < _NL - 1)
        def _coord_update():
            t = _relu(_dot(m, xW1_ref[l]) + xb1_ref[l].reshape(1, _H))
            tsq = _dot(t, xW2_ref[l]).reshape(_B, _IBS, _N)
            for c in range(4):
                xd = (xci_ref[k - 1, c][:, :, None]
                      - xc_ref[c][:, None, :])
                trans = jnp.clip(xd * tsq, -100.0, 100.0)
                aggx_ref[k - 1, c] = jnp.sum(trans, axis=2) / float(_N - 1)

    @pl.when(k == _NBLK)
    def _layer_epilogue():
        h = h_ref[...]
        hagg = hagg_ref[...].reshape(_B * _N, _H)
        scal2 = scalars_ref[...].reshape(_B * _N, _NS)
        hW1 = hW1_ref[l]
        hh1 = (_dot(h, hW1[0:_H, :]) + _dot(hagg, hW1[_H:2 * _H, :])
               + _dot(scal2, hW1[2 * _H:2 * _H + _NS, :])
               + hb1_ref[l].reshape(1, _H))
        mu = jnp.sum(hh1, axis=0).reshape(1, _H) / float(_B * _N)
        vr = (jnp.sum(hh1 * hh1, axis=0).reshape(1, _H) / float(_B * _N)
              - mu * mu)
        scn = hg_ref[l].reshape(1, _H) / jnp.sqrt(vr + _EPS)
        shn = hbe_ref[l].reshape(1, _H) - mu * scn
        hh = _relu(hh1 * scn + shn)
        hnew = h + _dot(hh, hW2_ref[l]) + hb2_ref[l].reshape(1, _H)
        h_ref[...] = hnew

        @pl.when(l < _NL - 1)
        def _apply_x():
            for c in range(4):
                agg = jnp.concatenate(
                    [aggx_ref[kk, c] for kk in range(_NBLK)], axis=-1)
                xf = xc_ref[c] + agg * _CW
                xc_ref[c] = xf
                for kk in range(_NBLK):
                    xci_ref[kk, c] = xf[:, kk * _IBS:(kk + 1) * _IBS]

        @pl.when(l == _NL - 1)
        def _decode():
            hm = jnp.sum(hnew.reshape(_B, _N, _H), axis=1) / float(_N)
            d1 = _relu(_dot(hm, dW1_ref[...]) + db1_ref[...])
            out_ref[...] = _dot(d1, dW2_ref[...]) + db2_ref[...]


def _const_spec(a):
    nd = a.ndim
    return pl.BlockSpec(a.shape, lambda l, k, _nd=nd: (0,) * _nd)


@jax.jit
def kernel(scalars, x, params):
    p = params
    args = (scalars, x,
            p['emb_W'], p['emb_b'].reshape(1, _H),
            p['phi_e_W1'], p['phi_e_g1'], p['phi_e_be1'],
            p['phi_e_W2'], p['phi_e_b2'],
            p['phi_m_W'], p['phi_m_b'],
            p['phi_x_W1'], p['phi_x_b1'], p['phi_x_W2'],
            p['phi_h_W1'], p['phi_h_b1'], p['phi_h_g'], p['phi_h_be'],
            p['phi_h_W2'], p['phi_h_b2'],
            p['dec_W1'], p['dec_b1'].reshape(1, _H),
            p['dec_W2'], p['dec_b2'].reshape(1, _NC))
    return pl.pallas_call(
        _lorentz_body,
        grid=(_NL, _NBLK + 1),
        in_specs=[_const_spec(a) for a in args],
        out_specs=pl.BlockSpec((_B, _NC), lambda l, k: (0, 0)),
        out_shape=jax.ShapeDtypeStruct((_B, _NC), jnp.float32),
        scratch_shapes=[
            pltpu.VMEM((_B * _N, _H), jnp.float32),
            pltpu.VMEM((4, _B, _N), jnp.float32),
            pltpu.VMEM((_NBLK, 4, _B, _IBS), jnp.float32),
            pltpu.VMEM((_B, _N, _N), jnp.float32),
            pltpu.VMEM((_B, _N, _N), jnp.float32),
            pltpu.VMEM((_B, _N, _H), jnp.float32),
            pltpu.VMEM((_B, _N, _H), jnp.float32),
            pltpu.VMEM((1, _H), jnp.float32),
            pltpu.VMEM((1, _H), jnp.float32),
            pltpu.VMEM((_B, _N, _H), jnp.float32),
            pltpu.VMEM((_NBLK, 4, _B, _IBS), jnp.float32),
        ],
        compiler_params=pltpu.CompilerParams(
            dimension_semantics=("arbitrary", "arbitrary")),
    )(*args)


# speedup vs baseline: 11.9247x; 1.0261x over previous
import jax
import jax.numpy as jnp
from jax.experimental import pallas as pl
from jax.experimental.pallas import tpu as pltpu

_NS = 7
_H = 72
_NC = 2
_NL = 6
_CW = 0.001
_B = 32
_N = 64
_IBS = 16
_NBLK = _N // _IBS
_EPS = 1e-5
_ECNT = float(_B * _N * (_N - 1))


def _psi(t):
    return jnp.sign(t) * jnp.log(jnp.abs(t) + 1.0)


def _relu(t):
    return jnp.maximum(t, 0.0)


def _dot(a, b):
    return jnp.dot(a, b, preferred_element_type=jnp.float32)


def _lorentz_body(scalars_ref, x_ref, emb_W_ref, emb_b_ref,
                  eW1_ref, eg1_ref, ebe1_ref, eW2_ref, eb2_ref,
                  mW_ref, mb_ref,
                  xW1_ref, xb1_ref, xW2_ref,
                  hW1_ref, hb1_ref, hg_ref, hbe_ref, hW2_ref, hb2_ref,
                  dW1_ref, db1_ref, dW2_ref, db2_ref,
                  out_ref,
                  h_ref, xc_ref, xci_ref, norms_ref, dots_ref, Ai_ref, Aj_ref,
                  wns_ref, wds_ref, hagg_ref, aggx_ref):
    l = pl.program_id(0)
    k = pl.program_id(1)
    i0 = (k - 1) * _IBS

    @pl.when(jnp.logical_and(l == 0, k == 0))
    def _init():
        scal2 = scalars_ref[...].reshape(_B * _N, _NS)
        h_ref[...] = _dot(scal2, emb_W_ref[...]) + emb_b_ref[...]
        for c in range(4):
            xf = x_ref[..., c]
            xc_ref[c] = xf
            for kk in range(_NBLK):
                xci_ref[kk, c] = xf[:, kk * _IBS:(kk + 1) * _IBS]

    @pl.when(k == 0)
    def _layer_prologue():
        x0, x1, x2, x3 = xc_ref[0], xc_ref[1], xc_ref[2], xc_ref[3]
        G = (x0[:, :, None] * x0[:, None, :]
             - x1[:, :, None] * x1[:, None, :]
             - x2[:, :, None] * x2[:, None, :]
             - x3[:, :, None] * x3[:, None, :])
        dG = x0 * x0 - x1 * x1 - x2 * x2 - x3 * x3
        dots = _psi(G)
        norms = _psi(dG[:, :, None] + dG[:, None, :] - 2.0 * G)
        dots_ref[...] = dots
        norms_ref[...] = norms
        w1 = eW1_ref[l]
        h = h_ref[...]
        Ai = _dot(h, w1[0:_H, :]).reshape(_B, _N, _H)
        Aj = _dot(h, w1[_H:2 * _H, :]).reshape(_B, _N, _H)

        iN = jax.lax.broadcasted_iota(jnp.int32, (_N, _N), 0)
        jN = jax.lax.broadcasted_iota(jnp.int32, (_N, _N), 1)
        mskNN = (iN != jN).astype(jnp.float32)[None, :, :]
        nm = norms * mskNN
        dm = dots * mskNN
        w_n = w1[2 * _H, :].reshape(1, _H)
        w_d = w1[2 * _H + 1, :].reshape(1, _H)

        SA = jnp.sum(Ai, axis=1)
        SB = jnp.sum(Aj, axis=1)
        QA = jnp.sum(Ai * Ai, axis=1)
        QB = jnp.sum(Aj * Aj, axis=1)
        PAB = jnp.sum(Ai * Aj, axis=1)
        nrow = jnp.sum(nm, axis=2)
        ncol = jnp.sum(nm, axis=1)
        drow = jnp.sum(dm, axis=2)
        dcol = jnp.sum(dm, axis=1)
        Vn = jnp.sum(Ai * nrow[..., None], axis=1)
        VnB = jnp.sum(Aj * ncol[..., None], axis=1)
        Vd = jnp.sum(Ai * drow[..., None], axis=1)
        VdB = jnp.sum(Aj * dcol[..., None], axis=1)
        Sn = jnp.sum(nrow)
        Sd = jnp.sum(drow)
        Snn = jnp.sum(nm * nm)
        Sdd = jnp.sum(dm * dm)
        Snd = jnp.sum(nm * dm)

        s1 = (float(_N - 1) * jnp.sum(SA + SB, axis=0).reshape(1, _H)
              + Sn * w_n + Sd * w_d)
        s2 = (float(_N - 1) * jnp.sum(QA + QB, axis=0).reshape(1, _H)
              + Snn * (w_n * w_n) + Sdd * (w_d * w_d)
              + 2.0 * jnp.sum(SA * SB - PAB, axis=0).reshape(1, _H)
              + 2.0 * w_n * jnp.sum(Vn + VnB, axis=0).reshape(1, _H)
              + 2.0 * w_d * jnp.sum(Vd + VdB, axis=0).reshape(1, _H)
              + 2.0 * Snd * (w_n * w_d))
        mean = s1 / _ECNT
        var = s2 / _ECNT - mean * mean
        scv = eg1_ref[l].reshape(1, _H) / jnp.sqrt(var + _EPS)
        shv = ebe1_ref[l].reshape(1, _H) - mean * scv
        Ai_ref[...] = Ai * scv.reshape(1, 1, _H) + shv.reshape(1, 1, _H)
        Aj_ref[...] = Aj * scv.reshape(1, 1, _H)
        wns_ref[...] = w_n * scv
        wds_ref[...] = w_d * scv

    @pl.when(k > 0)
    def _compute_pass():
        w_n = wns_ref[...].reshape(1, 1, 1, _H)
        w_d = wds_ref[...].reshape(1, 1, 1, _H)
        nb = norms_ref[:, pl.ds(i0, _IBS), :][..., None]
        db = dots_ref[:, pl.ds(i0, _IBS), :][..., None]
        ai = Ai_ref[:, pl.ds(i0, _IBS), :]
        m1 = (ai[:, :, None, :] + Aj_ref[...][:, None, :, :]
              + nb * w_n + db * w_d)

        ib = jax.lax.broadcasted_iota(jnp.int32, (_IBS, _N), 0) + i0
        jb = jax.lax.broadcasted_iota(jnp.int32, (_IBS, _N), 1)
        msk = (ib != jb).astype(jnp.float32)[None, :, :, None]

        e = _relu(m1).reshape(_B * _IBS * _N, _H)
        f = _relu(_dot(e, eW2_ref[l]) + eb2_ref[l].reshape(1, _H))
        wgt = jax.nn.sigmoid(_dot(f, mW_ref[l]) + mb_ref[l].reshape(1, 1))
        m = f * wgt
        m4 = m.reshape(_B, _IBS, _N, _H) * msk
        hagg_ref[:, pl.ds(i0, _IBS), :] = jnp.sum(m4, axis=2)

        @pl.when(l < _NL - 1)
        def _coord_update():
            t = _relu(_dot(m, xW1_ref[l]) + xb1_ref[l].reshape(1, _H))
            tsq = _dot(t, xW2_ref[l]).reshape(_B, _IBS, _N)
            for c in range(4):
                xd = (xci_ref[k - 1, c][:, :, None]
                      - xc_ref[c][:, None, :])
                trans = jnp.clip(xd * tsq, -100.0, 100.0)
                aggx_ref[k - 1, c] = jnp.sum(trans, axis=2) / float(_N - 1)

    @pl.when(k == _NBLK)
    def _layer_epilogue():
        h = h_ref[...]
        hagg = hagg_ref[...].reshape(_B * _N, _H)
        scal2 = scalars_ref[...].reshape(_B * _N, _NS)
        hW1 = hW1_ref[l]
        hh1 = (_dot(h, hW1[0:_H, :]) + _dot(hagg, hW1[_H:2 * _H, :])
               + _dot(scal2, hW1[2 * _H:2 * _H + _NS, :])
               + hb1_ref[l].reshape(1, _H))
        mu = jnp.sum(hh1, axis=0).reshape(1, _H) / float(_B * _N)
        vr = (jnp.sum(hh1 * hh1, axis=0).reshape(1, _H) / float(_B * _N)
              - mu * mu)
        scn = hg_ref[l].reshape(1, _H) / jnp.sqrt(vr + _EPS)
        shn = hbe_ref[l].reshape(1, _H) - mu * scn
        hh = _relu(hh1 * scn + shn)
        hnew = h + _dot(hh, hW2_ref[l]) + hb2_ref[l].reshape(1, _H)
        h_ref[...] = hnew

        @pl.when(l < _NL - 1)
        def _apply_x():
            for c in range(4):
                agg = jnp.concatenate(
                    [aggx_ref[kk, c] for kk in range(_NBLK)], axis=-1)
                xf = xc_ref[c] + agg * _CW
                xc_ref[c] = xf
                for kk in range(_NBLK):
                    xci_ref[kk, c] = xf[:, kk * _IBS:(kk + 1) * _IBS]

        @pl.when(l == _NL - 1)
        def _decode():
            hm = jnp.sum(hnew.reshape(_B, _N, _H), axis=1) / float(_N)
            d1 = _relu(_dot(hm, dW1_ref[...]) + db1_ref[...])
            out_ref[...] = _dot(d1, dW2_ref[...]) + db2_ref[...]


def _const_spec(a):
    nd = a.ndim
    return pl.BlockSpec(a.shape, lambda l, k, _nd=nd: (0,) * _nd)


@jax.jit
def kernel(scalars, x, params):
    p = params
    args = (scalars, x,
            p['emb_W'], p['emb_b'].reshape(1, _H),
            p['phi_e_W1'], p['phi_e_g1'], p['phi_e_be1'],
            p['phi_e_W2'], p['phi_e_b2'],
            p['phi_m_W'], p['phi_m_b'],
            p['phi_x_W1'], p['phi_x_b1'], p['phi_x_W2'],
            p['phi_h_W1'], p['phi_h_b1'], p['phi_h_g'], p['phi_h_be'],
            p['phi_h_W2'], p['phi_h_b2'],
            p['dec_W1'], p['dec_b1'].reshape(1, _H),
            p['dec_W2'], p['dec_b2'].reshape(1, _NC))
    return pl.pallas_call(
        _lorentz_body,
        grid=(_NL, _NBLK + 1),
        in_specs=[_const_spec(a) for a in args],
        out_specs=pl.BlockSpec((_B, _NC), lambda l, k: (0, 0)),
        out_shape=jax.ShapeDtypeStruct((_B, _NC), jnp.float32),
        scratch_shapes=[
            pltpu.VMEM((_B * _N, _H), jnp.float32),
            pltpu.VMEM((4, _B, _N), jnp.float32),
            pltpu.VMEM((_NBLK, 4, _B, _IBS), jnp.float32),
            pltpu.VMEM((_B, _N, _N), jnp.float32),
            pltpu.VMEM((_B, _N, _N), jnp.float32),
            pltpu.VMEM((_B, _N, _H), jnp.float32),
            pltpu.VMEM((_B, _N, _H), jnp.float32),
            pltpu.VMEM((1, _H), jnp.float32),
            pltpu.VMEM((1, _H), jnp.float32),
            pltpu.VMEM((_B, _N, _H), jnp.float32),
            pltpu.VMEM((_NBLK, 4, _B, _IBS), jnp.float32),
        ],
        compiler_params=pltpu.CompilerParams(
            dimension_semantics=("arbitrary", "arbitrary")),
    )(*args)
